# Initial kernel scaffold; baseline (speedup 1.0000x reference)
import jax
import jax.numpy as jnp
from jax.experimental import pallas as pl


def kernel(student_hidden, teacher_expert_outputs, W1, b1, g1, be1, W2, b2, g2, be2, W3, b3, Wp, bp, Wg1, bg1, gg, beg, Wg2, bg2, temperature, Ws1, bs1, Ws2, bs2, Wr, br):
    raise NotImplementedError("write your pallas kernel here")


# speedup vs baseline: 1.7422x; 1.7422x over previous
import functools

import jax
import jax.numpy as jnp
from jax.experimental import pallas as pl
from jax.experimental.pallas import tpu as pltpu

SD = 1024
TD = 2048
E = 8
K = 2
S = 2048
BLK = 256

_PREC = jax.lax.Precision.DEFAULT


def _dot(a, b):
    return jax.lax.dot_general(a, b, (((1,), (0,)), ((), ())),
                               precision=_PREC, preferred_element_type=jnp.float32)


def _ln(x, g, b):
    m = jnp.mean(x, axis=-1, keepdims=True)
    v = jnp.mean((x - m) ** 2, axis=-1, keepdims=True)
    return (x - m) / jnp.sqrt(v + 1e-5) * g + b


def _gelu(x):
    return 0.5 * x * (1.0 + jax.lax.erf(x * 0.7071067811865476))


def _cap_body(x_ref, w1_ref, b1_ref, g1_ref, be1_ref, w2_ref, b2_ref, g2_ref,
              be2_ref, w3_ref, b3_ref, t_ref, out_ref):
    x = x_ref[...]
    h = _gelu(_ln(_dot(x, w1_ref[...]) + b1_ref[...], g1_ref[...], be1_ref[...]))
    h = _gelu(_ln(_dot(h, w2_ref[...]) + b2_ref[...], g2_ref[...], be2_ref[...]))
    logits = _dot(h, w3_ref[...]) + b3_ref[...]
    out_ref[...] = jax.nn.softmax(logits / t_ref[...], axis=-1)


def _cap_scores(x, W1, b1, g1, be1, W2, b2, g2, be2, W3, b3, temperature):
    full = lambda shape: pl.BlockSpec(shape, lambda i: (0, 0))
    return pl.pallas_call(
        _cap_body,
        grid=(S // BLK,),
        in_specs=[
            pl.BlockSpec((BLK, SD), lambda i: (i, 0)),
            full((SD, 2 * SD)), full((1, 2 * SD)), full((1, 2 * SD)), full((1, 2 * SD)),
            full((2 * SD, SD)), full((1, SD)), full((1, SD)), full((1, SD)),
            full((SD, E)), full((1, E)), full((1, 1)),
        ],
        out_specs=pl.BlockSpec((BLK, E), lambda i: (i, 0)),
        out_shape=jax.ShapeDtypeStruct((S, E), jnp.float32),
    )(x, W1, b1.reshape(1, -1), g1.reshape(1, -1), be1.reshape(1, -1),
      W2, b2.reshape(1, -1), g2.reshape(1, -1), be2.reshape(1, -1),
      W3, b3.reshape(1, -1), temperature.reshape(1, 1))


def _gap_body(x_ref, t_ref, wp_ref, bp_ref, wg1_ref, bg1_ref, gg_ref, beg_ref,
              wg2_ref, bg2_ref, out_ref):
    x = x_ref[...]
    tproj = _dot(t_ref[...], wp_ref[...]) + bp_ref[...]
    pre = _dot(x, wg1_ref[0:SD, :]) + _dot(tproj, wg1_ref[SD:2 * SD, :]) + bg1_ref[...]
    gh = _gelu(_ln(pre, gg_ref[...], beg_ref[...]))
    out_ref[...] = jax.nn.softmax(_dot(gh, wg2_ref[...]) + bg2_ref[...], axis=-1)


def _gap_scores(x, t, Wp, bp, Wg1, bg1, gg, beg, Wg2, bg2):
    full = lambda shape: pl.BlockSpec(shape, lambda i: (0, 0))
    return pl.pallas_call(
        _gap_body,
        grid=(S // BLK,),
        in_specs=[
            pl.BlockSpec((BLK, SD), lambda i: (i, 0)),
            pl.BlockSpec((BLK, TD), lambda i: (i, 0)),
            full((TD, SD)), full((1, SD)),
            full((2 * SD, TD)), full((1, TD)), full((1, TD)), full((1, TD)),
            full((TD, E)), full((1, E)),
        ],
        out_specs=pl.BlockSpec((BLK, E), lambda i: (i, 0)),
        out_shape=jax.ShapeDtypeStruct((S, E), jnp.float32),
    )(x, t, Wp, bp.reshape(1, -1), Wg1, bg1.reshape(1, -1), gg.reshape(1, -1),
      beg.reshape(1, -1), Wg2, bg2.reshape(1, -1))


def _sel_body(t_ref, ws1_ref, bs1_ref, ws2_ref, bs2_ref, cap_ref, gap_ref,
              i0_ref, i1_ref, w0_ref, w1_ref, rw_ref):
    ei = jax.nn.softmax(
        _dot(_gelu(_dot(t_ref[...], ws1_ref[...]) + bs1_ref[...]), ws2_ref[...])
        + bs2_ref[...], axis=-1)
    comb = 0.4 * cap_ref[...] + 0.3 * gap_ref[...] + 0.3 * ei
    lane = jax.lax.broadcasted_iota(jnp.int32, (BLK, E), 1)
    m1 = jnp.max(comb, axis=-1, keepdims=True)
    i1 = jnp.min(jnp.where(comb == m1, lane, E), axis=-1, keepdims=True)
    masked = jnp.where(lane == i1, -jnp.inf, comb)
    m2 = jnp.max(masked, axis=-1, keepdims=True)
    i2 = jnp.min(jnp.where(masked == m2, lane, E), axis=-1, keepdims=True)
    denom = m1 + m2 + 1e-8
    w0 = m1 / denom
    w1 = m2 / denom
    tok = pl.program_id(0) * BLK + jax.lax.broadcasted_iota(jnp.int32, (BLK, 1), 0)
    i0_ref[...] = i1 * S + tok
    i1_ref[...] = i2 * S + tok
    w0_ref[...] = w0
    w1_ref[...] = w1
    rw = jnp.where(lane == i1, w0, 0.0) + jnp.where(lane == i2, w1, 0.0)
    rw_ref[...] = rw


def _select(t, Ws1, bs1, Ws2, bs2, cap_s, gap_s):
    full = lambda shape: pl.BlockSpec(shape, lambda i: (0, 0))
    blk_col = lambda n: pl.BlockSpec((BLK, n), lambda i: (i, 0))
    return pl.pallas_call(
        _sel_body,
        grid=(S // BLK,),
        in_specs=[
            pl.BlockSpec((BLK, TD), lambda i: (i, 0)),
            full((TD, TD // 2)), full((1, TD // 2)),
            full((TD // 2, E)), full((1, E)),
            blk_col(E), blk_col(E),
        ],
        out_specs=[blk_col(1), blk_col(1), blk_col(1), blk_col(1), blk_col(E)],
        out_shape=[
            jax.ShapeDtypeStruct((S, 1), jnp.int32),
            jax.ShapeDtypeStruct((S, 1), jnp.int32),
            jax.ShapeDtypeStruct((S, 1), jnp.float32),
            jax.ShapeDtypeStruct((S, 1), jnp.float32),
            jax.ShapeDtypeStruct((S, E), jnp.float32),
        ],
    )(t, Ws1, bs1.reshape(1, -1), Ws2, bs2.reshape(1, -1), cap_s, gap_s)


def _comb_body(stack_ref, rw_ref, out_ref):
    rw = rw_ref[...]
    acc = rw[:, 0:1] * stack_ref[0]
    for e in range(1, E):
        acc = acc + rw[:, e:e + 1] * stack_ref[e]
    out_ref[...] = acc


def _combine_dense(stack, rw):
    return pl.pallas_call(
        _comb_body,
        grid=(S // BLK,),
        in_specs=[
            pl.BlockSpec((E, BLK, TD), lambda i: (0, i, 0)),
            pl.BlockSpec((BLK, E), lambda i: (i, 0)),
        ],
        out_specs=pl.BlockSpec((BLK, TD), lambda i: (i, 0)),
        out_shape=jax.ShapeDtypeStruct((S, TD), jnp.float32),
    )(stack, rw)


def kernel(student_hidden, teacher_expert_outputs, W1, b1, g1, be1, W2, b2, g2,
           be2, W3, b3, Wp, bp, Wg1, bg1, gg, beg, Wg2, bg2, temperature,
           Ws1, bs1, Ws2, bs2, Wr, br):
    x = student_hidden.reshape(S, SD)
    teacher = teacher_expert_outputs[0].reshape(S, TD)
    cap_s = _cap_scores(x, W1, b1, g1, be1, W2, b2, g2, be2, W3, b3, temperature)
    gap_s = _gap_scores(x, teacher, Wp, bp, Wg1, bg1, gg, beg, Wg2, bg2)
    i0, i1, w0, w1, rw = _select(teacher, Ws1, bs1, Ws2, bs2, cap_s, gap_s)
    stack = teacher_expert_outputs.reshape(E, S, TD)
    routed = _combine_dense(stack, rw)
    return routed.reshape(1, S, TD)
